# baseline (speedup 1.0000x reference)
import functools

import jax
import jax.numpy as jnp
from jax import lax
from jax.experimental import pallas as pl
from jax.experimental.pallas import tpu as pltpu
from jax.experimental.pallas import tpu_sc as plsc

N = 10000
E = 160000
H = 128
NB = 8
R_MAX = 5.0
P = 5.0
AVG_NEIGH = 16.0
SH = 9

NC = 2
NS = 16
NW = NC * NS

N_PAD = 10240
E_PAD = 163840

_S3 = 3.0 ** 0.5
_S5 = 5.0 ** 0.5
_S15 = 15.0 ** 0.5

_mesh = plsc.VectorSubcoreMesh(
    core_axis_name="c", subcore_axis_name="s", num_cores=NC, num_subcores=NS)


def _sc_gather(table, idx2d, width):
  rows = idx2d.shape[0] * 128
  dt = table.dtype
  nb = rows // (NW * 128)
  nbuf = 4 if width >= 128 else 2
  depth = nbuf // 2

  def body(table_ref, idx_ref, out_ref, idx_v, rows_v, sem, sem_w):
    wid = lax.axis_index("s") * NC + lax.axis_index("c")
    b0 = wid * nb
    pltpu.sync_copy(idx_ref.at[pl.ds(b0, nb)], idx_v)
    for p in range(depth):
      pltpu.async_copy(table_ref.at[idx_v.at[p]], rows_v.at[p], sem)

    @pl.loop(0, nb, step=nbuf)
    def _g(g):
      for b in range(nbuf):
        j = g + b
        pltpu.make_async_copy(
            table_ref.at[idx_v.at[j]], rows_v.at[b], sem).wait()
        pltpu.async_copy(
            rows_v.at[b], out_ref.at[pl.ds((b0 + j) * 128, 128)], sem_w)

        @pl.when(j >= depth)
        def _():
          pltpu.make_async_copy(
              rows_v.at[(b + depth) % nbuf],
              out_ref.at[pl.ds(b0 * 128, 128)], sem_w).wait()

        @pl.when(j + depth < nb)
        def _():
          pltpu.async_copy(
              table_ref.at[idx_v.at[j + depth]],
              rows_v.at[(b + depth) % nbuf], sem)

    for p in range(depth):
      pltpu.make_async_copy(
          rows_v.at[p], out_ref.at[pl.ds(b0 * 128, 128)], sem_w).wait()

  f = pl.kernel(
      body,
      out_type=jax.ShapeDtypeStruct((rows, width), dt),
      mesh=_mesh,
      scratch_types=[
          pltpu.VMEM((nb, 128), jnp.int32),
          pltpu.VMEM((nbuf, 128, width), dt),
          pltpu.SemaphoreType.DMA,
          pltpu.SemaphoreType.DMA,
      ],
      compiler_params=pltpu.CompilerParams(use_tc_tiling_on_sc=False),
  )
  return f(table, idx2d)


def _sc_scatter(msg, dst2d, init):
  rows = dst2d.shape[0] * 128
  nb = rows // (NS * 128)
  rows_t = N_PAD // NS
  full_init = init.ndim == 3

  def body(msg_ref, idx_ref, z_ref, acc_ref, A, idx_v, buf, sem, sem_w):
    c = lax.axis_index("c")
    s = lax.axis_index("s")
    pltpu.sync_copy(idx_ref.at[pl.ds(s * nb, nb)], idx_v)
    for cc in range(5):
      if cc < 4:
        chunk = 2 * cc + c
        slot = chunk
        j0 = 0
        npass = nb
      else:
        chunk = jnp.int32(8)
        slot = 8 + c
        j0 = c * (nb // 2)
        npass = nb // 2

      if full_init:
        pltpu.sync_copy(z_ref.at[slot, pl.ds(s * rows_t, rows_t)],
                        A.at[pl.ds(s * rows_t, rows_t)])
      else:
        pltpu.sync_copy(z_ref, A.at[pl.ds(s * rows_t, rows_t)])
      plsc.subcore_barrier()
      pltpu.async_copy(
          msg_ref.at[chunk, pl.ds((s * nb + j0) * 128, 128)], buf.at[0], sem)

      @pl.loop(0, npass, step=2)
      def _g(g):
        for b in range(2):
          lj = g + b
          j = j0 + lj
          pltpu.make_async_copy(
              msg_ref.at[chunk, pl.ds((s * nb + j) * 128, 128)],
              buf.at[b], sem).wait()
          pltpu.async_copy(buf.at[b], A.at[idx_v.at[j]], sem_w, add=True)

          @pl.when(lj >= 1)
          def _():
            pltpu.make_async_copy(
                buf.at[1 - b], A.at[idx_v.at[j]], sem_w).wait()

          @pl.when(lj + 1 < npass)
          def _():
            pltpu.async_copy(
                msg_ref.at[chunk, pl.ds((s * nb + j + 1) * 128, 128)],
                buf.at[1 - b], sem)

      pltpu.make_async_copy(buf.at[0], A.at[idx_v.at[j0]], sem_w).wait()
      plsc.subcore_barrier()
      pltpu.sync_copy(
          A.at[pl.ds(s * rows_t, rows_t)],
          acc_ref.at[slot, pl.ds(s * rows_t, rows_t)])
      plsc.subcore_barrier()

  f = pl.kernel(
      body,
      out_type=jax.ShapeDtypeStruct((SH + 1, N_PAD, 128), jnp.float32),
      mesh=_mesh,
      scratch_types=[
          pltpu.VMEM_SHARED((N_PAD, 128), jnp.float32),
          pltpu.VMEM((nb, 128), jnp.int32),
          pltpu.VMEM((2, 128, 128), jnp.float32),
          pltpu.SemaphoreType.DMA,
          pltpu.SemaphoreType.DMA,
      ],
  )
  return f(msg, dst2d, init)


def _tc_embed(node_attrs_pad, W_emb, aE_row):
  bn = 512
  grid = N_PAD // bn

  def body(na_ref, we_ref, ae_ref, h_ref, e_ref):
    na = na_ref[...]
    h_ref[...] = lax.dot_general(
        na, we_ref[...], (((1,), (0,)), ((), ())),
        preferred_element_type=jnp.float32).astype(jnp.bfloat16)
    e_ref[...] = jnp.sum(na * ae_ref[...], axis=1, keepdims=True)

  return pl.pallas_call(
      body,
      grid=(grid,),
      in_specs=[
          pl.BlockSpec((bn, 10), lambda i: (i, 0)),
          pl.BlockSpec((10, H), lambda i: (0, 0)),
          pl.BlockSpec((1, 10), lambda i: (0, 0)),
      ],
      out_specs=[
          pl.BlockSpec((bn, H), lambda i: (i, 0)),
          pl.BlockSpec((bn, 1), lambda i: (i, 0)),
      ],
      out_shape=[
          jax.ShapeDtypeStruct((N_PAD, H), jnp.bfloat16),
          jax.ShapeDtypeStruct((N_PAD, 1), jnp.float32),
      ],
  )(node_attrs_pad, W_emb, aE_row)


def _silu(x):
  return x * jax.nn.sigmoid(x)


def _tc_edges(psd, sh, h_src, rw1, rb1, rw2, rb2, rw3p, rb3p, e_off, e_cnt):
  be = 512
  grid = e_cnt // be
  off = e_off // be
  doff = E_PAD // be

  def body(ps_ref, pd_ref, sh_ref, hs_ref, w1, b1, w2, b2, w3, b3, out_ref):
    d = pd_ref[...] - ps_ref[...] + sh_ref[...]
    dT = d.T
    x = dT[0:1, :]
    y = dT[1:2, :]
    z = dT[2:3, :]
    r2 = x * x + y * y + z * z
    r = jnp.sqrt(r2 + 1e-12)
    inv_r = 1.0 / r
    x = x * inv_r
    y = y * inv_r
    z = z * inv_r

    n = lax.broadcasted_iota(jnp.int32, (NB, 1), 0).astype(jnp.float32) + 1.0
    bes = (2.0 / R_MAX) ** 0.5 * jnp.sin(n * (jnp.pi / R_MAX) * r) * inv_r
    xr = r * (1.0 / R_MAX)
    x2 = xr * xr
    x5 = x2 * x2 * xr
    env = (1.0 - 0.5 * (P + 1.0) * (P + 2.0) * x5
           + P * (P + 2.0) * x5 * xr
           - 0.5 * P * (P + 1.0) * x5 * x2)
    env = jnp.where(xr < 1.0, env, 0.0)
    efT = bes * env

    ytop = jnp.concatenate(
        [_S3 * x, _S3 * y, _S3 * z,
         _S15 * x * y, _S15 * y * z, 0.5 * _S5 * (3.0 * z * z - 1.0),
         _S15 * x * z, 0.5 * _S15 * (x * x - y * y)], axis=0)
    ycols = ytop.T
    ef = efT.T

    t = _silu(lax.dot_general(ef, w1[...], (((1,), (0,)), ((), ())),
                              preferred_element_type=jnp.float32) + b1[...])
    t = _silu(lax.dot_general(t, w2[...], (((1,), (0,)), ((), ())),
                              preferred_element_type=jnp.float32) + b2[...])
    R = lax.dot_general(t, w3[...], (((1,), (0,)), ((), ())),
                        preferred_element_type=jnp.float32) + b3[...]

    hs = hs_ref[...].astype(jnp.float32)
    P0 = hs * R[:, 0:H]
    P1 = hs * R[:, H:2 * H]
    P2 = hs * R[:, 2 * H:3 * H]

    out_ref[0] = P0
    for cidx in range(1, 4):
      out_ref[cidx] = P1 * ycols[:, cidx - 1:cidx]
    for cidx in range(4, SH):
      out_ref[cidx] = P2 * ycols[:, cidx - 1:cidx]

  return pl.pallas_call(
      body,
      grid=(grid,),
      in_specs=[
          pl.BlockSpec((be, 16), lambda i: (i + off, 0)),
          pl.BlockSpec((be, 16), lambda i: (i + off + doff, 0)),
          pl.BlockSpec((be, 16), lambda i: (i + off, 0)),
          pl.BlockSpec((be, H), lambda i: (i, 0)),
          pl.BlockSpec((NB, 64), lambda i: (0, 0)),
          pl.BlockSpec((1, 64), lambda i: (0, 0)),
          pl.BlockSpec((64, 64), lambda i: (0, 0)),
          pl.BlockSpec((1, 64), lambda i: (0, 0)),
          pl.BlockSpec((64, 3 * H), lambda i: (0, 0)),
          pl.BlockSpec((1, 3 * H), lambda i: (0, 0)),
      ],
      out_specs=pl.BlockSpec((SH, be, H), lambda i: (0, i, 0)),
      out_shape=jax.ShapeDtypeStruct((SH, e_cnt, H), jnp.float32),
  )(psd, psd, sh, h_src, rw1, rb1, rw2, rb2, rw3p, rb3p)


def _tc_nodes(acc, e_in, WpT, Wh, Wr1, Wr2r):
  bn = 512
  grid = N_PAD // bn

  def body(a_ref, e_ref, wp_ref, wh_ref, wr1_ref, wr2_ref, h_ref, eo_ref,
           tot_ref):
    A = a_ref[...] * (1.0 / AVG_NEIGH)
    a0 = A[0]
    a8 = A[8] + A[9]
    l1 = A[1] * A[1] + A[2] * A[2] + A[3] * A[3]
    l2 = (A[4] * A[4] + A[5] * A[5] + A[6] * A[6]
          + A[7] * A[7] + a8 * a8)
    wp = wp_ref[...]
    B = a0 * wp[0:1, :] + l1 * wp[1:2, :] + l2 * wp[2:3, :]
    h = _silu(lax.dot_general(B, wh_ref[...], (((1,), (0,)), ((), ())),
                              preferred_element_type=jnp.float32))
    t = _silu(lax.dot_general(h, wr1_ref[...], (((1,), (0,)), ((), ())),
                              preferred_element_type=jnp.float32))
    de = jnp.sum(t * wr2_ref[...], axis=1, keepdims=True)
    nid = (pl.program_id(0) * bn
           + lax.broadcasted_iota(jnp.int32, (bn, 1), 0))
    de = jnp.where(nid < N, de, 0.0)
    h_ref[...] = h.astype(jnp.bfloat16)
    eo = e_ref[...] + de
    eo_ref[...] = eo

    @pl.when(pl.program_id(0) == 0)
    def _():
      tot_ref[...] = jnp.zeros_like(tot_ref)

    tot_ref[...] += jnp.sum(eo, keepdims=True)

  return pl.pallas_call(
      body,
      grid=(grid,),
      in_specs=[
          pl.BlockSpec((SH + 1, bn, H), lambda i: (0, i, 0)),
          pl.BlockSpec((bn, 1), lambda i: (i, 0)),
          pl.BlockSpec((3, H), lambda i: (0, 0)),
          pl.BlockSpec((H, H), lambda i: (0, 0)),
          pl.BlockSpec((H, 16), lambda i: (0, 0)),
          pl.BlockSpec((1, 16), lambda i: (0, 0)),
      ],
      out_specs=[
          pl.BlockSpec((bn, H), lambda i: (i, 0)),
          pl.BlockSpec((bn, 1), lambda i: (i, 0)),
          pl.BlockSpec((1, 1), lambda i: (0, 0)),
      ],
      out_shape=[
          jax.ShapeDtypeStruct((N_PAD, H), jnp.bfloat16),
          jax.ShapeDtypeStruct((N_PAD, 1), jnp.float32),
          jax.ShapeDtypeStruct((1, 1), jnp.float32),
      ],
  )(acc, e_in, WpT, Wh, Wr1, Wr2r)


def _tc_total(e):
  bn = 2048
  grid = N_PAD // bn

  def body(e_ref, out_ref):
    @pl.when(pl.program_id(0) == 0)
    def _():
      out_ref[...] = jnp.zeros_like(out_ref)
    out_ref[...] += jnp.sum(e_ref[...], keepdims=True)

  return pl.pallas_call(
      body,
      grid=(grid,),
      in_specs=[pl.BlockSpec((bn, 1), lambda i: (i, 0))],
      out_specs=pl.BlockSpec((1, 1), lambda i: (0, 0)),
      out_shape=jax.ShapeDtypeStruct((1, 1), jnp.float32),
  )(e)


def kernel(positions, node_attrs, edge_index, shifts, batch, W_emb, atomic_E,
           rw1, rb1, rw2, rb2, rw3, rb3, Wp, Wh, Wr1, Wr2):
  f32 = jnp.float32
  src = edge_index[0].astype(jnp.int32)
  dst = edge_index[1].astype(jnp.int32)
  pad_e = E_PAD - E
  src_pad = jnp.concatenate([src, jnp.zeros((pad_e,), jnp.int32)])
  dst_pad = jnp.concatenate(
      [dst, N + (jnp.arange(pad_e, dtype=jnp.int32) % (N_PAD - N))])
  src2d = src_pad.reshape(E_PAD // 128, 128)
  dst2d = dst_pad.reshape(E_PAD // 128, 128)
  dstg2d = jnp.where(dst2d < N, dst2d, 0)

  pos16 = jnp.pad(positions.astype(f32), ((0, 0), (0, 13)))
  sh16 = jnp.pad(shifts.astype(f32), ((0, 0), (0, 13)))
  sh16 = jnp.pad(sh16, ((0, pad_e), (0, 0)))
  na_pad = jnp.pad(node_attrs.astype(f32), ((0, N_PAD - N), (0, 0)))
  zeros_init = jnp.zeros((N_PAD // NS, 128), f32)
  aE_row = atomic_E.reshape(1, -1).astype(f32)

  psd = _sc_gather(pos16, jnp.concatenate([src2d, dstg2d], axis=0), 16)

  h, e = _tc_embed(na_pad, W_emb.astype(f32), aE_row)

  EA = 49152
  BA = EA // 128
  src_a, src_b = src2d[:BA], src2d[BA:]
  dst_a, dst_b = dst2d[:BA], dst2d[BA:]

  L = rw1.shape[0]
  total = None
  for i in range(L):
    rw3p = rw3[i].reshape(64, H, 3).transpose(0, 2, 1).reshape(64, 3 * H)
    rb3p = rb3[i].reshape(H, 3).T.reshape(1, 3 * H)
    w = (rw1[i], rb1[i].reshape(1, 64), rw2[i], rb2[i].reshape(1, 64),
         rw3p, rb3p)
    hs_a = _sc_gather(h, src_a, H)
    hs_b = _sc_gather(h, src_b, H)
    msg_a = _tc_edges(psd, sh16, hs_a, *w, 0, EA)
    acc_a = _sc_scatter(msg_a, dst_a, zeros_init)
    msg_b = _tc_edges(psd, sh16, hs_b, *w, EA, E_PAD - EA)
    acc = _sc_scatter(msg_b, dst_b, acc_a)
    h, e, total = _tc_nodes(acc, e, Wp[i].T, Wh[i], Wr1[i],
                            Wr2[i].reshape(1, 16))

  return total.reshape(1)


# speedup vs baseline: 1.0185x; 1.0185x over previous
import functools

import jax
import jax.numpy as jnp
from jax import lax
from jax.experimental import pallas as pl
from jax.experimental.pallas import tpu as pltpu
from jax.experimental.pallas import tpu_sc as plsc

N = 10000
E = 160000
H = 128
NB = 8
R_MAX = 5.0
P = 5.0
AVG_NEIGH = 16.0
SH = 9

NC = 2
NS = 16
NW = NC * NS

N_PAD = 10240
E_PAD = 163840

_S3 = 3.0 ** 0.5
_S5 = 5.0 ** 0.5
_S15 = 15.0 ** 0.5

_mesh = plsc.VectorSubcoreMesh(
    core_axis_name="c", subcore_axis_name="s", num_cores=NC, num_subcores=NS)


def _sc_gather(table, idx2d, width):
  rows = idx2d.shape[0] * 128
  dt = table.dtype
  nb = rows // (NW * 128)
  nbuf = 4 if width >= 128 else 2
  depth = nbuf // 2

  def body(table_ref, idx_ref, out_ref, idx_v, rows_v, sem, sem_w):
    wid = lax.axis_index("s") * NC + lax.axis_index("c")
    b0 = wid * nb
    pltpu.sync_copy(idx_ref.at[pl.ds(b0, nb)], idx_v)
    for p in range(depth):
      pltpu.async_copy(table_ref.at[idx_v.at[p]], rows_v.at[p], sem)

    @pl.loop(0, nb, step=nbuf)
    def _g(g):
      for b in range(nbuf):
        j = g + b
        pltpu.make_async_copy(
            table_ref.at[idx_v.at[j]], rows_v.at[b], sem).wait()
        pltpu.async_copy(
            rows_v.at[b], out_ref.at[pl.ds((b0 + j) * 128, 128)], sem_w)

        @pl.when(j >= depth)
        def _():
          pltpu.make_async_copy(
              rows_v.at[(b + depth) % nbuf],
              out_ref.at[pl.ds(b0 * 128, 128)], sem_w).wait()

        @pl.when(j + depth < nb)
        def _():
          pltpu.async_copy(
              table_ref.at[idx_v.at[j + depth]],
              rows_v.at[(b + depth) % nbuf], sem)

    for p in range(depth):
      pltpu.make_async_copy(
          rows_v.at[p], out_ref.at[pl.ds(b0 * 128, 128)], sem_w).wait()

  f = pl.kernel(
      body,
      out_type=jax.ShapeDtypeStruct((rows, width), dt),
      mesh=_mesh,
      scratch_types=[
          pltpu.VMEM((nb, 128), jnp.int32),
          pltpu.VMEM((nbuf, 128, width), dt),
          pltpu.SemaphoreType.DMA,
          pltpu.SemaphoreType.DMA,
      ],
      compiler_params=pltpu.CompilerParams(use_tc_tiling_on_sc=False),
  )
  return f(table, idx2d)


def _sc_scatter(msg, dst2d, init):
  rows = dst2d.shape[0] * 128
  nb = rows // (NS * 128)
  rows_t = N_PAD // NS
  full_init = init.ndim == 3

  def body(msg_ref, idx_ref, z_ref, acc_ref, A, idx_v, buf, sem, sem_w):
    c = lax.axis_index("c")
    s = lax.axis_index("s")
    pltpu.sync_copy(idx_ref.at[pl.ds(s * nb, nb)], idx_v)
    for cc in range(5):
      if cc < 4:
        chunk = 2 * cc + c
        slot = chunk
        j0 = 0
        npass = nb
      else:
        chunk = jnp.int32(8)
        slot = 8 + c
        j0 = c * (nb // 2)
        npass = nb // 2

      if full_init:
        pltpu.sync_copy(z_ref.at[slot, pl.ds(s * rows_t, rows_t)],
                        A.at[pl.ds(s * rows_t, rows_t)])
      else:
        pltpu.sync_copy(z_ref, A.at[pl.ds(s * rows_t, rows_t)])
      plsc.subcore_barrier()
      pltpu.async_copy(
          msg_ref.at[chunk, pl.ds((s * nb + j0) * 128, 128)], buf.at[0], sem)

      @pl.loop(0, npass, step=2)
      def _g(g):
        for b in range(2):
          lj = g + b
          j = j0 + lj
          pltpu.make_async_copy(
              msg_ref.at[chunk, pl.ds((s * nb + j) * 128, 128)],
              buf.at[b], sem).wait()
          pltpu.async_copy(buf.at[b], A.at[idx_v.at[j]], sem_w, add=True)

          @pl.when(lj >= 1)
          def _():
            pltpu.make_async_copy(
                buf.at[1 - b], A.at[idx_v.at[j]], sem_w).wait()

          @pl.when(lj + 1 < npass)
          def _():
            pltpu.async_copy(
                msg_ref.at[chunk, pl.ds((s * nb + j + 1) * 128, 128)],
                buf.at[1 - b], sem)

      pltpu.make_async_copy(buf.at[0], A.at[idx_v.at[j0]], sem_w).wait()
      plsc.subcore_barrier()
      pltpu.sync_copy(
          A.at[pl.ds(s * rows_t, rows_t)],
          acc_ref.at[slot, pl.ds(s * rows_t, rows_t)])
      plsc.subcore_barrier()

  f = pl.kernel(
      body,
      out_type=jax.ShapeDtypeStruct((SH + 1, N_PAD, 128), jnp.float32),
      mesh=_mesh,
      scratch_types=[
          pltpu.VMEM_SHARED((N_PAD, 128), jnp.float32),
          pltpu.VMEM((nb, 128), jnp.int32),
          pltpu.VMEM((2, 128, 128), jnp.float32),
          pltpu.SemaphoreType.DMA,
          pltpu.SemaphoreType.DMA,
      ],
  )
  return f(msg, dst2d, init)


def _tc_embed(node_attrs_pad, W_emb, aE_row):
  bn = 512
  grid = N_PAD // bn

  def body(na_ref, we_ref, ae_ref, h_ref, e_ref):
    na = na_ref[...]
    h_ref[...] = lax.dot_general(
        na, we_ref[...], (((1,), (0,)), ((), ())),
        preferred_element_type=jnp.float32)
    e_ref[...] = jnp.sum(na * ae_ref[...], axis=1, keepdims=True)

  return pl.pallas_call(
      body,
      grid=(grid,),
      in_specs=[
          pl.BlockSpec((bn, 10), lambda i: (i, 0)),
          pl.BlockSpec((10, H), lambda i: (0, 0)),
          pl.BlockSpec((1, 10), lambda i: (0, 0)),
      ],
      out_specs=[
          pl.BlockSpec((bn, H), lambda i: (i, 0)),
          pl.BlockSpec((bn, 1), lambda i: (i, 0)),
      ],
      out_shape=[
          jax.ShapeDtypeStruct((N_PAD, H), jnp.float32),
          jax.ShapeDtypeStruct((N_PAD, 1), jnp.float32),
      ],
  )(node_attrs_pad, W_emb, aE_row)


def _silu(x):
  return x * jax.nn.sigmoid(x)


def _tc_edges(psd, sh, h_src, rw1, rb1, rw2, rb2, rw3p, rb3p, e_off, e_cnt):
  be = 512
  grid = e_cnt // be
  off = e_off // be
  doff = E_PAD // be

  def body(ps_ref, pd_ref, sh_ref, hs_ref, w1, b1, w2, b2, w3, b3, out_ref):
    d = pd_ref[...] - ps_ref[...] + sh_ref[...]
    dT = d.T
    x = dT[0:1, :]
    y = dT[1:2, :]
    z = dT[2:3, :]
    r2 = x * x + y * y + z * z
    r = jnp.sqrt(r2 + 1e-12)
    inv_r = 1.0 / r
    x = x * inv_r
    y = y * inv_r
    z = z * inv_r

    n = lax.broadcasted_iota(jnp.int32, (NB, 1), 0).astype(jnp.float32) + 1.0
    bes = (2.0 / R_MAX) ** 0.5 * jnp.sin(n * (jnp.pi / R_MAX) * r) * inv_r
    xr = r * (1.0 / R_MAX)
    x2 = xr * xr
    x5 = x2 * x2 * xr
    env = (1.0 - 0.5 * (P + 1.0) * (P + 2.0) * x5
           + P * (P + 2.0) * x5 * xr
           - 0.5 * P * (P + 1.0) * x5 * x2)
    env = jnp.where(xr < 1.0, env, 0.0)
    efT = bes * env

    ytop = jnp.concatenate(
        [_S3 * x, _S3 * y, _S3 * z,
         _S15 * x * y, _S15 * y * z, 0.5 * _S5 * (3.0 * z * z - 1.0),
         _S15 * x * z, 0.5 * _S15 * (x * x - y * y)], axis=0)
    ycols = ytop.T
    ef = efT.T

    t = _silu(lax.dot_general(ef, w1[...], (((1,), (0,)), ((), ())),
                              preferred_element_type=jnp.float32) + b1[...])
    t = _silu(lax.dot_general(t, w2[...], (((1,), (0,)), ((), ())),
                              preferred_element_type=jnp.float32) + b2[...])
    R = lax.dot_general(t, w3[...], (((1,), (0,)), ((), ())),
                        preferred_element_type=jnp.float32) + b3[...]

    hs = hs_ref[...]
    P0 = hs * R[:, 0:H]
    P1 = hs * R[:, H:2 * H]
    P2 = hs * R[:, 2 * H:3 * H]

    out_ref[0] = P0
    for cidx in range(1, 4):
      out_ref[cidx] = P1 * ycols[:, cidx - 1:cidx]
    for cidx in range(4, SH):
      out_ref[cidx] = P2 * ycols[:, cidx - 1:cidx]

  return pl.pallas_call(
      body,
      grid=(grid,),
      in_specs=[
          pl.BlockSpec((be, 16), lambda i: (i + off, 0)),
          pl.BlockSpec((be, 16), lambda i: (i + off + doff, 0)),
          pl.BlockSpec((be, 16), lambda i: (i + off, 0)),
          pl.BlockSpec((be, H), lambda i: (i, 0)),
          pl.BlockSpec((NB, 64), lambda i: (0, 0)),
          pl.BlockSpec((1, 64), lambda i: (0, 0)),
          pl.BlockSpec((64, 64), lambda i: (0, 0)),
          pl.BlockSpec((1, 64), lambda i: (0, 0)),
          pl.BlockSpec((64, 3 * H), lambda i: (0, 0)),
          pl.BlockSpec((1, 3 * H), lambda i: (0, 0)),
      ],
      out_specs=pl.BlockSpec((SH, be, H), lambda i: (0, i, 0)),
      out_shape=jax.ShapeDtypeStruct((SH, e_cnt, H), jnp.float32),
  )(psd, psd, sh, h_src, rw1, rb1, rw2, rb2, rw3p, rb3p)


def _tc_nodes(acc, e_in, WpT, Wh, Wr1, Wr2r):
  bn = 512
  grid = N_PAD // bn

  def body(a_ref, e_ref, wp_ref, wh_ref, wr1_ref, wr2_ref, h_ref, eo_ref,
           tot_ref):
    A = a_ref[...] * (1.0 / AVG_NEIGH)
    a0 = A[0]
    a8 = A[8] + A[9]
    l1 = A[1] * A[1] + A[2] * A[2] + A[3] * A[3]
    l2 = (A[4] * A[4] + A[5] * A[5] + A[6] * A[6]
          + A[7] * A[7] + a8 * a8)
    wp = wp_ref[...]
    B = a0 * wp[0:1, :] + l1 * wp[1:2, :] + l2 * wp[2:3, :]
    h = _silu(lax.dot_general(B, wh_ref[...], (((1,), (0,)), ((), ())),
                              preferred_element_type=jnp.float32))
    t = _silu(lax.dot_general(h, wr1_ref[...], (((1,), (0,)), ((), ())),
                              preferred_element_type=jnp.float32))
    de = jnp.sum(t * wr2_ref[...], axis=1, keepdims=True)
    nid = (pl.program_id(0) * bn
           + lax.broadcasted_iota(jnp.int32, (bn, 1), 0))
    de = jnp.where(nid < N, de, 0.0)
    h_ref[...] = h
    eo = e_ref[...] + de
    eo_ref[...] = eo

    @pl.when(pl.program_id(0) == 0)
    def _():
      tot_ref[...] = jnp.zeros_like(tot_ref)

    tot_ref[...] += jnp.sum(eo, keepdims=True)

  return pl.pallas_call(
      body,
      grid=(grid,),
      in_specs=[
          pl.BlockSpec((SH + 1, bn, H), lambda i: (0, i, 0)),
          pl.BlockSpec((bn, 1), lambda i: (i, 0)),
          pl.BlockSpec((3, H), lambda i: (0, 0)),
          pl.BlockSpec((H, H), lambda i: (0, 0)),
          pl.BlockSpec((H, 16), lambda i: (0, 0)),
          pl.BlockSpec((1, 16), lambda i: (0, 0)),
      ],
      out_specs=[
          pl.BlockSpec((bn, H), lambda i: (i, 0)),
          pl.BlockSpec((bn, 1), lambda i: (i, 0)),
          pl.BlockSpec((1, 1), lambda i: (0, 0)),
      ],
      out_shape=[
          jax.ShapeDtypeStruct((N_PAD, H), jnp.float32),
          jax.ShapeDtypeStruct((N_PAD, 1), jnp.float32),
          jax.ShapeDtypeStruct((1, 1), jnp.float32),
      ],
  )(acc, e_in, WpT, Wh, Wr1, Wr2r)


def _tc_total(e):
  bn = 2048
  grid = N_PAD // bn

  def body(e_ref, out_ref):
    @pl.when(pl.program_id(0) == 0)
    def _():
      out_ref[...] = jnp.zeros_like(out_ref)
    out_ref[...] += jnp.sum(e_ref[...], keepdims=True)

  return pl.pallas_call(
      body,
      grid=(grid,),
      in_specs=[pl.BlockSpec((bn, 1), lambda i: (i, 0))],
      out_specs=pl.BlockSpec((1, 1), lambda i: (0, 0)),
      out_shape=jax.ShapeDtypeStruct((1, 1), jnp.float32),
  )(e)


def kernel(positions, node_attrs, edge_index, shifts, batch, W_emb, atomic_E,
           rw1, rb1, rw2, rb2, rw3, rb3, Wp, Wh, Wr1, Wr2):
  f32 = jnp.float32
  src = edge_index[0].astype(jnp.int32)
  dst = edge_index[1].astype(jnp.int32)
  pad_e = E_PAD - E
  src_pad = jnp.concatenate([src, jnp.zeros((pad_e,), jnp.int32)])
  dst_pad = jnp.concatenate(
      [dst, N + (jnp.arange(pad_e, dtype=jnp.int32) % (N_PAD - N))])
  src2d = src_pad.reshape(E_PAD // 128, 128)
  dst2d = dst_pad.reshape(E_PAD // 128, 128)
  dstg2d = jnp.where(dst2d < N, dst2d, 0)

  pos16 = jnp.pad(positions.astype(f32), ((0, 0), (0, 13)))
  sh16 = jnp.pad(shifts.astype(f32), ((0, 0), (0, 13)))
  sh16 = jnp.pad(sh16, ((0, pad_e), (0, 0)))
  na_pad = jnp.pad(node_attrs.astype(f32), ((0, N_PAD - N), (0, 0)))
  zeros_init = jnp.zeros((N_PAD // NS, 128), f32)
  aE_row = atomic_E.reshape(1, -1).astype(f32)

  psd = _sc_gather(pos16, jnp.concatenate([src2d, dstg2d], axis=0), 16)

  h, e = _tc_embed(na_pad, W_emb.astype(f32), aE_row)

  EA = 32768
  BA = EA // 128
  src_a, src_b = src2d[:BA], src2d[BA:]
  dst_a, dst_b = dst2d[:BA], dst2d[BA:]

  L = rw1.shape[0]
  total = None
  for i in range(L):
    rw3p = rw3[i].reshape(64, H, 3).transpose(0, 2, 1).reshape(64, 3 * H)
    rb3p = rb3[i].reshape(H, 3).T.reshape(1, 3 * H)
    w = (rw1[i], rb1[i].reshape(1, 64), rw2[i], rb2[i].reshape(1, 64),
         rw3p, rb3p)
    hs_a = _sc_gather(h, src_a, H)
    hs_b = _sc_gather(h, src_b, H)
    msg_a = _tc_edges(psd, sh16, hs_a, *w, 0, EA)
    acc_a = _sc_scatter(msg_a, dst_a, zeros_init)
    msg_b = _tc_edges(psd, sh16, hs_b, *w, EA, E_PAD - EA)
    acc = _sc_scatter(msg_b, dst_b, acc_a)
    h, e, total = _tc_nodes(acc, e, Wp[i].T, Wh[i], Wr1[i],
                            Wr2[i].reshape(1, 16))

  return total.reshape(1)


# speedup vs baseline: 1.0523x; 1.0332x over previous
import jax
import jax.numpy as jnp
from jax import lax
from jax.experimental import pallas as pl
from jax.experimental.pallas import tpu as pltpu
from jax.experimental.pallas import tpu_sc as plsc

N = 10000
E = 160000
H = 128
NB = 8
R_MAX = 5.0
P = 5.0
AVG_NEIGH = 16.0
SH = 9

NC = 2
NS = 16
NW = NC * NS

N_PAD = 10240
E_PAD = 163840

_S3 = 3.0 ** 0.5
_S5 = 5.0 ** 0.5
_S15 = 15.0 ** 0.5

_mesh = plsc.VectorSubcoreMesh(
    core_axis_name="c", subcore_axis_name="s", num_cores=NC, num_subcores=NS)


def _sc_gather(table, idx2d, width):
  rows = idx2d.shape[0] * 128
  dt = table.dtype
  nb = rows // (NW * 128)
  nbuf = 4 if width >= 128 else 2
  depth = nbuf // 2

  def body(table_ref, idx_ref, out_ref, idx_v, rows_v, sem, sem_w):
    wid = lax.axis_index("s") * NC + lax.axis_index("c")
    b0 = wid * nb
    pltpu.sync_copy(idx_ref.at[pl.ds(b0, nb)], idx_v)
    for p in range(depth):
      pltpu.async_copy(table_ref.at[idx_v.at[p]], rows_v.at[p], sem)

    @pl.loop(0, nb, step=nbuf)
    def _g(g):
      for b in range(nbuf):
        j = g + b
        pltpu.make_async_copy(
            table_ref.at[idx_v.at[j]], rows_v.at[b], sem).wait()
        pltpu.async_copy(
            rows_v.at[b], out_ref.at[pl.ds((b0 + j) * 128, 128)], sem_w)

        @pl.when(j >= depth)
        def _():
          pltpu.make_async_copy(
              rows_v.at[(b + depth) % nbuf],
              out_ref.at[pl.ds(b0 * 128, 128)], sem_w).wait()

        @pl.when(j + depth < nb)
        def _():
          pltpu.async_copy(
              table_ref.at[idx_v.at[j + depth]],
              rows_v.at[(b + depth) % nbuf], sem)

    for p in range(depth):
      pltpu.make_async_copy(
          rows_v.at[p], out_ref.at[pl.ds(b0 * 128, 128)], sem_w).wait()

  f = pl.kernel(
      body,
      out_type=jax.ShapeDtypeStruct((rows, width), dt),
      mesh=_mesh,
      scratch_types=[
          pltpu.VMEM((nb, 128), jnp.int32),
          pltpu.VMEM((nbuf, 128, width), dt),
          pltpu.SemaphoreType.DMA,
          pltpu.SemaphoreType.DMA,
      ],
      compiler_params=pltpu.CompilerParams(use_tc_tiling_on_sc=False),
  )
  return f(table, idx2d)


def _sc_scatter(msg, dst2d, init):
  rows = dst2d.shape[0] * 128
  nb = rows // (NS * 128)
  rows_t = N_PAD // NS
  full_init = init.ndim == 3

  def body(msg_ref, idx_ref, z_ref, acc_ref, A, idx_v, buf, sem, sem_w):
    c = lax.axis_index("c")
    s = lax.axis_index("s")
    pltpu.sync_copy(idx_ref.at[pl.ds(s * nb, nb)], idx_v)
    for cc in range(5):
      if cc < 4:
        chunk = 2 * cc + c
        slot = chunk
        j0 = 0
        npass = nb
      else:
        chunk = jnp.int32(8)
        slot = 8 + c
        j0 = c * (nb // 2)
        npass = nb // 2

      if full_init:
        pltpu.sync_copy(z_ref.at[slot, pl.ds(s * rows_t, rows_t)],
                        A.at[pl.ds(s * rows_t, rows_t)])
      else:
        pltpu.sync_copy(z_ref, A.at[pl.ds(s * rows_t, rows_t)])
      plsc.subcore_barrier()
      pltpu.async_copy(
          msg_ref.at[chunk, pl.ds((s * nb + j0) * 128, 128)], buf.at[0], sem)

      @pl.loop(0, npass, step=2)
      def _g(g):
        for b in range(2):
          lj = g + b
          j = j0 + lj
          pltpu.make_async_copy(
              msg_ref.at[chunk, pl.ds((s * nb + j) * 128, 128)],
              buf.at[b], sem).wait()
          pltpu.async_copy(buf.at[b], A.at[idx_v.at[j]], sem_w, add=True)

          @pl.when(lj >= 1)
          def _():
            pltpu.make_async_copy(
                buf.at[1 - b], A.at[idx_v.at[j]], sem_w).wait()

          @pl.when(lj + 1 < npass)
          def _():
            pltpu.async_copy(
                msg_ref.at[chunk, pl.ds((s * nb + j + 1) * 128, 128)],
                buf.at[1 - b], sem)

      pltpu.make_async_copy(buf.at[0], A.at[idx_v.at[j0]], sem_w).wait()
      plsc.subcore_barrier()
      pltpu.sync_copy(
          A.at[pl.ds(s * rows_t, rows_t)],
          acc_ref.at[slot, pl.ds(s * rows_t, rows_t)])
      plsc.subcore_barrier()

  f = pl.kernel(
      body,
      out_type=jax.ShapeDtypeStruct((SH + 1, N_PAD, 128), jnp.float32),
      mesh=_mesh,
      scratch_types=[
          pltpu.VMEM_SHARED((N_PAD, 128), jnp.float32),
          pltpu.VMEM((nb, 128), jnp.int32),
          pltpu.VMEM((2, 128, 128), jnp.float32),
          pltpu.SemaphoreType.DMA,
          pltpu.SemaphoreType.DMA,
      ],
  )
  return f(msg, dst2d, init)


def _tc_embed(node_attrs_pad, W_emb, aE_row):
  bn = 512
  grid = N_PAD // bn

  def body(na_ref, we_ref, ae_ref, h_ref, e_ref):
    na = na_ref[...]
    h_ref[...] = lax.dot_general(
        na, we_ref[...], (((1,), (0,)), ((), ())),
        preferred_element_type=jnp.float32)
    e_ref[...] = jnp.sum(na * ae_ref[...], axis=1, keepdims=True)

  return pl.pallas_call(
      body,
      grid=(grid,),
      in_specs=[
          pl.BlockSpec((bn, 10), lambda i: (i, 0)),
          pl.BlockSpec((10, H), lambda i: (0, 0)),
          pl.BlockSpec((1, 10), lambda i: (0, 0)),
      ],
      out_specs=[
          pl.BlockSpec((bn, H), lambda i: (i, 0)),
          pl.BlockSpec((bn, 1), lambda i: (i, 0)),
      ],
      out_shape=[
          jax.ShapeDtypeStruct((N_PAD, H), jnp.float32),
          jax.ShapeDtypeStruct((N_PAD, 1), jnp.float32),
      ],
  )(node_attrs_pad, W_emb, aE_row)


def _silu(x):
  return x * jax.nn.sigmoid(x)


def _tc_edges(psd, sh, h_src, rw1, rb1, rw2, rb2, rw3p, rb3p, e_off, e_cnt):
  be = 512
  grid = e_cnt // be
  off = e_off // be
  doff = E_PAD // be

  def body(ps_ref, pd_ref, sh_ref, hs_ref, w1, b1, w2, b2, w3, b3, out_ref):
    d = pd_ref[...] - ps_ref[...] + sh_ref[...]
    dT = d.T
    x = dT[0:1, :]
    y = dT[1:2, :]
    z = dT[2:3, :]
    r2 = x * x + y * y + z * z
    r = jnp.sqrt(r2 + 1e-12)
    inv_r = 1.0 / r
    x = x * inv_r
    y = y * inv_r
    z = z * inv_r

    n = lax.broadcasted_iota(jnp.int32, (NB, 1), 0).astype(jnp.float32) + 1.0
    bes = (2.0 / R_MAX) ** 0.5 * jnp.sin(n * (jnp.pi / R_MAX) * r) * inv_r
    xr = r * (1.0 / R_MAX)
    x2 = xr * xr
    x5 = x2 * x2 * xr
    env = (1.0 - 0.5 * (P + 1.0) * (P + 2.0) * x5
           + P * (P + 2.0) * x5 * xr
           - 0.5 * P * (P + 1.0) * x5 * x2)
    env = jnp.where(xr < 1.0, env, 0.0)
    efT = bes * env

    ytop = jnp.concatenate(
        [_S3 * x, _S3 * y, _S3 * z,
         _S15 * x * y, _S15 * y * z, 0.5 * _S5 * (3.0 * z * z - 1.0),
         _S15 * x * z, 0.5 * _S15 * (x * x - y * y)], axis=0)
    ycols = ytop.T
    ef = efT.T

    t = _silu(lax.dot_general(ef, w1[...], (((1,), (0,)), ((), ())),
                              preferred_element_type=jnp.float32) + b1[...])
    t = _silu(lax.dot_general(t, w2[...], (((1,), (0,)), ((), ())),
                              preferred_element_type=jnp.float32) + b2[...])
    R = lax.dot_general(t, w3[...], (((1,), (0,)), ((), ())),
                        preferred_element_type=jnp.float32) + b3[...]

    hs = hs_ref[...]
    P0 = hs * R[:, 0:H]
    P1 = hs * R[:, H:2 * H]
    P2 = hs * R[:, 2 * H:3 * H]

    out_ref[0] = P0
    for cidx in range(1, 4):
      out_ref[cidx] = P1 * ycols[:, cidx - 1:cidx]
    for cidx in range(4, SH):
      out_ref[cidx] = P2 * ycols[:, cidx - 1:cidx]

  return pl.pallas_call(
      body,
      grid=(grid,),
      in_specs=[
          pl.BlockSpec((be, 16), lambda i: (i + off, 0)),
          pl.BlockSpec((be, 16), lambda i: (i + off + doff, 0)),
          pl.BlockSpec((be, 16), lambda i: (i + off, 0)),
          pl.BlockSpec((be, H), lambda i: (i, 0)),
          pl.BlockSpec((NB, 64), lambda i: (0, 0)),
          pl.BlockSpec((1, 64), lambda i: (0, 0)),
          pl.BlockSpec((64, 64), lambda i: (0, 0)),
          pl.BlockSpec((1, 64), lambda i: (0, 0)),
          pl.BlockSpec((64, 3 * H), lambda i: (0, 0)),
          pl.BlockSpec((1, 3 * H), lambda i: (0, 0)),
      ],
      out_specs=pl.BlockSpec((SH, be, H), lambda i: (0, i, 0)),
      out_shape=jax.ShapeDtypeStruct((SH, e_cnt, H), jnp.float32),
  )(psd, psd, sh, h_src, rw1, rb1, rw2, rb2, rw3p, rb3p)


def _tc_nodes(acc, e_in, WpT, Wh, Wr1, Wr2r):
  bn = 512
  grid = N_PAD // bn

  def body(a_ref, e_ref, wp_ref, wh_ref, wr1_ref, wr2_ref, h_ref, eo_ref,
           tot_ref):
    A = a_ref[...] * (1.0 / AVG_NEIGH)
    a0 = A[0]
    a8 = A[8] + A[9]
    l1 = A[1] * A[1] + A[2] * A[2] + A[3] * A[3]
    l2 = (A[4] * A[4] + A[5] * A[5] + A[6] * A[6]
          + A[7] * A[7] + a8 * a8)
    wp = wp_ref[...]
    B = a0 * wp[0:1, :] + l1 * wp[1:2, :] + l2 * wp[2:3, :]
    h = _silu(lax.dot_general(B, wh_ref[...], (((1,), (0,)), ((), ())),
                              preferred_element_type=jnp.float32))
    t = _silu(lax.dot_general(h, wr1_ref[...], (((1,), (0,)), ((), ())),
                              preferred_element_type=jnp.float32))
    de = jnp.sum(t * wr2_ref[...], axis=1, keepdims=True)
    nid = (pl.program_id(0) * bn
           + lax.broadcasted_iota(jnp.int32, (bn, 1), 0))
    de = jnp.where(nid < N, de, 0.0)
    h_ref[...] = h
    eo = e_ref[...] + de
    eo_ref[...] = eo

    @pl.when(pl.program_id(0) == 0)
    def _():
      tot_ref[...] = jnp.zeros_like(tot_ref)

    tot_ref[...] += jnp.sum(eo, keepdims=True)

  return pl.pallas_call(
      body,
      grid=(grid,),
      in_specs=[
          pl.BlockSpec((SH + 1, bn, H), lambda i: (0, i, 0)),
          pl.BlockSpec((bn, 1), lambda i: (i, 0)),
          pl.BlockSpec((3, H), lambda i: (0, 0)),
          pl.BlockSpec((H, H), lambda i: (0, 0)),
          pl.BlockSpec((H, 16), lambda i: (0, 0)),
          pl.BlockSpec((1, 16), lambda i: (0, 0)),
      ],
      out_specs=[
          pl.BlockSpec((bn, H), lambda i: (i, 0)),
          pl.BlockSpec((bn, 1), lambda i: (i, 0)),
          pl.BlockSpec((1, 1), lambda i: (0, 0)),
      ],
      out_shape=[
          jax.ShapeDtypeStruct((N_PAD, H), jnp.float32),
          jax.ShapeDtypeStruct((N_PAD, 1), jnp.float32),
          jax.ShapeDtypeStruct((1, 1), jnp.float32),
      ],
  )(acc, e_in, WpT, Wh, Wr1, Wr2r)


def kernel(positions, node_attrs, edge_index, shifts, batch, W_emb, atomic_E,
           rw1, rb1, rw2, rb2, rw3, rb3, Wp, Wh, Wr1, Wr2):
  f32 = jnp.float32
  src = edge_index[0].astype(jnp.int32)
  dst = edge_index[1].astype(jnp.int32)
  pad_e = E_PAD - E
  src_pad = jnp.concatenate([src, jnp.zeros((pad_e,), jnp.int32)])
  dst_pad = jnp.concatenate(
      [dst, N + (jnp.arange(pad_e, dtype=jnp.int32) % (N_PAD - N))])
  src2d = src_pad.reshape(E_PAD // 128, 128)
  dst2d = dst_pad.reshape(E_PAD // 128, 128)
  dstg2d = jnp.where(dst2d < N, dst2d, 0)

  pos16 = jnp.pad(positions.astype(f32), ((0, 0), (0, 13)))
  sh16 = jnp.pad(shifts.astype(f32), ((0, 0), (0, 13)))
  sh16 = jnp.pad(sh16, ((0, pad_e), (0, 0)))
  na_pad = jnp.pad(node_attrs.astype(f32), ((0, N_PAD - N), (0, 0)))
  zeros_init = jnp.zeros((N_PAD // NS, 128), f32)
  aE_row = atomic_E.reshape(1, -1).astype(f32)

  psd = _sc_gather(pos16, jnp.concatenate([src2d, dstg2d], axis=0), 16)

  h, e = _tc_embed(na_pad, W_emb.astype(f32), aE_row)

  EA = 49152
  BA = EA // 128
  src_a, src_b = src2d[:BA], src2d[BA:]
  dst_a, dst_b = dst2d[:BA], dst2d[BA:]

  L = rw1.shape[0]
  total = None
  for i in range(L):
    rw3p = rw3[i].reshape(64, H, 3).transpose(0, 2, 1).reshape(64, 3 * H)
    rb3p = rb3[i].reshape(H, 3).T.reshape(1, 3 * H)
    w = (rw1[i], rb1[i].reshape(1, 64), rw2[i], rb2[i].reshape(1, 64),
         rw3p, rb3p)
    hs_a = _sc_gather(h, src_a, H)
    hs_b = _sc_gather(h, src_b, H)
    msg_a = _tc_edges(psd, sh16, hs_a, *w, 0, EA)
    acc_a = _sc_scatter(msg_a, dst_a, zeros_init)
    msg_b = _tc_edges(psd, sh16, hs_b, *w, EA, E_PAD - EA)
    acc = _sc_scatter(msg_b, dst_b, acc_a)
    h, e, total = _tc_nodes(acc, e, Wp[i].T, Wh[i], Wr1[i],
                            Wr2[i].reshape(1, 16))

  return total.reshape(1)
